# baseline (speedup 1.0000x reference)
import jax
import jax.numpy as jnp
import numpy as np
from jax import lax
from jax.experimental import pallas as pl
from jax.experimental.pallas import tpu as pltpu
from jax.experimental.pallas import tpu_sc as plsc

_MAXLEN = 8192
_DIM = 64
_BATCH = 64

_NC = 2
_NS = 16
_NW = _NC * _NS

_SLAB = _MAXLEN // _NW
_NCHUNK = _BATCH
_NBUF = 2
_LANES = 16
_VPR = _DIM // _LANES


def _sc_body(x_hbm, pos_hbm, out_hbm,
             bufs, pos_buf, lsem0, lsem1, ssem0, ssem1):
    lsems = (lsem0, lsem1)
    ssems = (ssem0, ssem1)

    wid = lax.axis_index("s") * _NC + lax.axis_index("c")
    base_pos = wid * _SLAB

    pltpu.sync_copy(pos_hbm.at[pl.ds(base_pos, _SLAB)], pos_buf)

    def row0(c):
        return c * _MAXLEN + base_pos

    def load(c, k):
        pltpu.async_copy(x_hbm.at[pl.ds(row0(c), _SLAB)], bufs.at[k],
                         lsems[k])

    def wait_load(c, k):
        pltpu.make_async_copy(x_hbm.at[pl.ds(row0(c), _SLAB)], bufs.at[k],
                              lsems[k]).wait()

    def store(c, k):
        pltpu.async_copy(bufs.at[k], out_hbm.at[pl.ds(row0(c), _SLAB)],
                         ssems[k])

    def wait_store(c, k):
        pltpu.make_async_copy(bufs.at[k], out_hbm.at[pl.ds(row0(c), _SLAB)],
                              ssems[k]).wait()

    load(0, 0)
    for c in range(_NCHUNK):
        k = c % _NBUF
        wait_load(c, k)

        @plsc.parallel_loop(0, _SLAB, unroll=4)
        def _(r):
            for li in range(_VPR):
                sl = pl.ds(li * _LANES, _LANES)
                plsc.addupdate(bufs.at[k, r, sl], pos_buf[r, sl])

        store(c, k)
        if c + 1 < _NCHUNK:
            if c >= 1:
                wait_store(c - 1, (k + 1) % _NBUF)
            load(c + 1, (k + 1) % _NBUF)
    wait_store(_NCHUNK - 1, (_NCHUNK - 1) % _NBUF)


_sc_call = pl.kernel(
    _sc_body,
    out_type=jax.ShapeDtypeStruct((_BATCH * _MAXLEN, _DIM), jnp.float32),
    mesh=plsc.VectorSubcoreMesh(core_axis_name="c", subcore_axis_name="s"),
    scratch_types=[
        pltpu.VMEM((_NBUF, _SLAB, _DIM), jnp.float32),
        pltpu.VMEM((_SLAB, _DIM), jnp.float32),
        pltpu.SemaphoreType.DMA,
        pltpu.SemaphoreType.DMA,
        pltpu.SemaphoreType.DMA,
        pltpu.SemaphoreType.DMA,
    ],
)


@jax.jit
def kernel(x, pos_table):
    out = _sc_call(x.reshape(_BATCH * _MAXLEN, _DIM), pos_table)
    return out.reshape(x.shape)


# speedup vs baseline: 1.1284x; 1.1284x over previous
import jax
import jax.numpy as jnp
import numpy as np
from jax import lax
from jax.experimental import pallas as pl
from jax.experimental.pallas import tpu as pltpu
from jax.experimental.pallas import tpu_sc as plsc

_MAXLEN = 8192
_DIM = 64
_BATCH = 64

_NC = 2
_NS = 16
_NW = _NC * _NS

_SLAB = _MAXLEN // _NW
_NCHUNK = _BATCH
_NBUF = 2
_LANES = 16
_VPR = _DIM // _LANES


def _sc_body(x_hbm, pos_hbm, out_hbm,
             bufs, pos_buf, lsem0, lsem1, ssem0, ssem1):
    lsems = (lsem0, lsem1)
    ssems = (ssem0, ssem1)

    wid = lax.axis_index("s") * _NC + lax.axis_index("c")
    base_pos = wid * _SLAB

    pltpu.sync_copy(pos_hbm.at[pl.ds(base_pos, _SLAB)], pos_buf)

    def row0(c):
        return c * _MAXLEN + base_pos

    def load(c, k):
        pltpu.async_copy(x_hbm.at[pl.ds(row0(c), _SLAB)], bufs.at[k],
                         lsems[k])

    def wait_load(c, k):
        pltpu.make_async_copy(x_hbm.at[pl.ds(row0(c), _SLAB)], bufs.at[k],
                              lsems[k]).wait()

    def store(c, k):
        pltpu.async_copy(bufs.at[k], out_hbm.at[pl.ds(row0(c), _SLAB)],
                         ssems[k])

    def wait_store(c, k):
        pltpu.make_async_copy(bufs.at[k], out_hbm.at[pl.ds(row0(c), _SLAB)],
                              ssems[k]).wait()

    load(0, 0)
    for c in range(_NCHUNK):
        k = c % _NBUF
        wait_load(c, k)
        if c + 1 < _NCHUNK:
            if c >= 1:
                wait_store(c - 1, (k + 1) % _NBUF)
            load(c + 1, (k + 1) % _NBUF)

        @plsc.parallel_loop(0, _SLAB, unroll=4)
        def _(r):
            for li in range(_VPR):
                sl = pl.ds(li * _LANES, _LANES)
                plsc.addupdate(bufs.at[k, r, sl], pos_buf[r, sl])

        store(c, k)
    wait_store(_NCHUNK - 1, (_NCHUNK - 1) % _NBUF)


_sc_call = pl.kernel(
    _sc_body,
    out_type=jax.ShapeDtypeStruct((_BATCH * _MAXLEN, _DIM), jnp.float32),
    mesh=plsc.VectorSubcoreMesh(core_axis_name="c", subcore_axis_name="s"),
    scratch_types=[
        pltpu.VMEM((_NBUF, _SLAB, _DIM), jnp.float32),
        pltpu.VMEM((_SLAB, _DIM), jnp.float32),
        pltpu.SemaphoreType.DMA,
        pltpu.SemaphoreType.DMA,
        pltpu.SemaphoreType.DMA,
        pltpu.SemaphoreType.DMA,
    ],
)


@jax.jit
def kernel(x, pos_table):
    out = _sc_call(x.reshape(_BATCH * _MAXLEN, _DIM), pos_table)
    return out.reshape(x.shape)
